# baseline (speedup 1.0000x reference)
import functools

import jax
import jax.numpy as jnp
from jax import lax
from jax.experimental import pallas as pl
from jax.experimental.pallas import tpu as pltpu
from jax.experimental.pallas import tpu_sc as plsc

_KC = 1024
_D = 128
_TOK = 2 * 16 * 2048
_T = 512
_NT = _TOK // _T
_NW = 32
_TPW = _TOK // _NW
_RPW = _TPW // 128


def _dist_body(z_ref, cbt2_ref, c2_ref, z2_ref, idx_ref, loss_ref):
    z = z_ref[...]
    cbt2 = cbt2_ref[...]
    logits2 = jnp.dot(z, cbt2, preferred_element_type=jnp.float32)
    z2 = z2_ref[...]
    c2 = c2_ref[...]
    dist = z2 + c2 + logits2
    m = jnp.min(dist, axis=1, keepdims=True)
    iota = lax.broadcasted_iota(jnp.int32, (_T, _KC), 1)
    idx_ref[0] = jnp.min(jnp.where(dist <= m, iota, _KC),
                         axis=1, keepdims=True)
    loss_ref[0, 0, 0] = jnp.sum(m)


def _nearest(z_flat, cbt2, c2, z2):
    idx, loss = pl.pallas_call(
        _dist_body,
        grid=(_NT,),
        in_specs=[
            pl.BlockSpec((_T, _D), lambda t: (t, 0)),
            pl.BlockSpec((_D, _KC), lambda t: (0, 0)),
            pl.BlockSpec((1, _KC), lambda t: (0, 0)),
            pl.BlockSpec((_T, 1), lambda t: (t, 0)),
        ],
        out_specs=[
            pl.BlockSpec((1, _T, 1), lambda t: (t, 0, 0)),
            pl.BlockSpec((1, 1, 1), lambda t: (t, 0, 0),
                         memory_space=pltpu.SMEM),
        ],
        out_shape=[
            jax.ShapeDtypeStruct((_NT, _T, 1), jnp.int32),
            jax.ShapeDtypeStruct((_NT, 1, 1), jnp.float32),
        ],
    )(z_flat, cbt2, c2, z2)
    return idx.reshape(_TOK // 128, 128), loss


def _gather_count_body(cb_ref, idx_ref, zq_ref, cnt_ref,
                       idx_v, rows0_v, rows1_v, cnt_v, sem0, sem1):
    c = lax.axis_index("c")
    s = lax.axis_index("s")
    wid = s * 2 + c
    base = wid * _TPW
    pltpu.sync_copy(idx_ref.at[pl.ds(wid * _RPW, _RPW)], idx_v)

    def _zero(i, carry):
        cnt_v[pl.ds(i * 16, 16)] = jnp.zeros((16,), jnp.float32)
        return carry

    lax.fori_loop(0, _KC // 16, _zero, 0)

    ones = jnp.ones((16,), jnp.float32)

    def _count_row(j):
        def _cnt(k, inner):
            iv = idx_v[j, pl.ds(k * 16, 16)]
            plsc.addupdate_scatter(cnt_v, [iv], ones)
            return inner

        lax.fori_loop(0, 8, _cnt, 0)

    def _row_pair(g, carry):
        j0 = 2 * g
        j1 = 2 * g + 1
        cp0 = pltpu.async_copy(cb_ref.at[idx_v.at[j0]], rows0_v, sem0)
        cp1 = pltpu.async_copy(cb_ref.at[idx_v.at[j1]], rows1_v, sem1)
        cp0.wait()
        pltpu.sync_copy(rows0_v, zq_ref.at[pl.ds(base + j0 * 128, 128)])
        cp1.wait()
        pltpu.sync_copy(rows1_v, zq_ref.at[pl.ds(base + j1 * 128, 128)])
        _count_row(j0)
        _count_row(j1)
        return carry

    lax.fori_loop(0, _RPW // 2, _row_pair, 0)
    pltpu.sync_copy(cnt_v, cnt_ref.at[wid])


@functools.cache
def _gather_count():
    mesh = plsc.VectorSubcoreMesh(core_axis_name="c", subcore_axis_name="s")
    return pl.kernel(
        _gather_count_body,
        mesh=mesh,
        out_type=[
            jax.ShapeDtypeStruct((_TOK, _D), jnp.float32),
            jax.ShapeDtypeStruct((_NW, _KC), jnp.float32),
        ],
        scratch_types=[
            pltpu.VMEM((_RPW, 128), jnp.int32),
            pltpu.VMEM((128, _D), jnp.float32),
            pltpu.VMEM((128, _D), jnp.float32),
            pltpu.VMEM((_KC,), jnp.float32),
            pltpu.SemaphoreType.DMA,
            pltpu.SemaphoreType.DMA,
        ],
        compiler_params=pltpu.CompilerParams(needs_layout_passes=False),
    )


def kernel(K, V, cb_k, cb_v, step):
    zk = K.reshape(_TOK, _D)
    zv = V.reshape(_TOK, _D)
    c2_k = jnp.sum(cb_k ** 2, axis=1)[None, :]
    c2_v = jnp.sum(cb_v ** 2, axis=1)[None, :]
    z2_k = jnp.sum(zk ** 2, axis=1, keepdims=True)
    z2_v = jnp.sum(zv ** 2, axis=1, keepdims=True)
    idx_k, loss_k = _nearest(zk, cb_k.T * -2.0, c2_k, z2_k)
    zq_k, cnt_k = _gather_count()(cb_k, idx_k)
    idx_vv, loss_v = _nearest(zv, cb_v.T * -2.0, c2_v, z2_v)
    zq_v, cnt_v = _gather_count()(cb_v, idx_vv)
    K_mix = zq_k.reshape(K.shape)
    V_mix = zq_v.reshape(V.shape)
    denom = float(_TOK * _D)
    lk = jnp.sum(loss_k) / denom
    lv = jnp.sum(loss_v) / denom
    usage_k = jnp.sum(cnt_k, axis=0) / float(_TOK)
    usage_v = jnp.sum(cnt_v, axis=0) / float(_TOK)
    return (K_mix, V_mix, 0.25 * lk, 0.25 * lv, 0.25 * lk, 0.25 * lv,
            usage_k, usage_v)


# speedup vs baseline: 1.3345x; 1.3345x over previous
import functools

import jax
import jax.numpy as jnp
from jax import lax
from jax.experimental import pallas as pl
from jax.experimental.pallas import tpu as pltpu
from jax.experimental.pallas import tpu_sc as plsc

_KC = 1024
_D = 128
_TOK = 2 * 16 * 2048
_T = 512
_NT = _TOK // _T
_NW = 32
_TPW = _TOK // _NW
_RPW = _TPW // 128


def _dist_body(z_ref, cb2_ref, c2_ref, z2_ref, idx_ref, loss_ref):
    z = z_ref[...]
    cb2 = cb2_ref[...]
    logits2 = lax.dot_general(cb2, z, (((1,), (1,)), ((), ())),
                              preferred_element_type=jnp.float32)
    z2 = z2_ref[0]
    c2 = c2_ref[...]
    dist = z2 + c2 + logits2
    m = jnp.min(dist, axis=0, keepdims=True)
    iota = lax.broadcasted_iota(jnp.int32, (_KC, _T), 0)
    idx_ref[0] = jnp.min(jnp.where(dist <= m, iota, _KC),
                         axis=0, keepdims=True)
    loss_ref[0, 0, 0] = jnp.sum(m)


def _nearest(z_flat, cb2, c2, z2):
    idx, loss = pl.pallas_call(
        _dist_body,
        grid=(_NT,),
        in_specs=[
            pl.BlockSpec((_T, _D), lambda t: (t, 0)),
            pl.BlockSpec((_KC, _D), lambda t: (0, 0)),
            pl.BlockSpec((_KC, 1), lambda t: (0, 0)),
            pl.BlockSpec((1, 1, _T), lambda t: (t, 0, 0)),
        ],
        out_specs=[
            pl.BlockSpec((1, 1, _T), lambda t: (t, 0, 0)),
            pl.BlockSpec((1, 1, 1), lambda t: (t, 0, 0),
                         memory_space=pltpu.SMEM),
        ],
        out_shape=[
            jax.ShapeDtypeStruct((_NT, 1, _T), jnp.int32),
            jax.ShapeDtypeStruct((_NT, 1, 1), jnp.float32),
        ],
    )(z_flat, cb2, c2, z2)
    return idx.reshape(_TOK // 128, 128), loss


def _gather_count_body(cb_ref, idx_ref, zq_ref, cnt_ref,
                       idx_v, rows0_v, rows1_v, cnt_v, sem0, sem1):
    c = lax.axis_index("c")
    s = lax.axis_index("s")
    wid = s * 2 + c
    base = wid * _TPW
    pltpu.sync_copy(idx_ref.at[pl.ds(wid * _RPW, _RPW)], idx_v)

    def _zero(i, carry):
        cnt_v[pl.ds(i * 16, 16)] = jnp.zeros((16,), jnp.float32)
        return carry

    lax.fori_loop(0, _KC // 16, _zero, 0)

    ones = jnp.ones((16,), jnp.float32)

    def _count_row(j):
        def _cnt(k, inner):
            iv = idx_v[j, pl.ds(k * 16, 16)]
            plsc.addupdate_scatter(cnt_v, [iv], ones)
            return inner

        lax.fori_loop(0, 8, _cnt, 0)

    def _row_pair(g, carry):
        j0 = 2 * g
        j1 = 2 * g + 1
        cp0 = pltpu.async_copy(cb_ref.at[idx_v.at[j0]], rows0_v, sem0)
        cp1 = pltpu.async_copy(cb_ref.at[idx_v.at[j1]], rows1_v, sem1)
        cp0.wait()
        pltpu.sync_copy(rows0_v, zq_ref.at[pl.ds(base + j0 * 128, 128)])
        cp1.wait()
        pltpu.sync_copy(rows1_v, zq_ref.at[pl.ds(base + j1 * 128, 128)])
        _count_row(j0)
        _count_row(j1)
        return carry

    lax.fori_loop(0, _RPW // 2, _row_pair, 0)
    pltpu.sync_copy(cnt_v, cnt_ref.at[wid])


@functools.cache
def _gather_count():
    mesh = plsc.VectorSubcoreMesh(core_axis_name="c", subcore_axis_name="s")
    return pl.kernel(
        _gather_count_body,
        mesh=mesh,
        out_type=[
            jax.ShapeDtypeStruct((_TOK, _D), jnp.float32),
            jax.ShapeDtypeStruct((_NW, _KC), jnp.float32),
        ],
        scratch_types=[
            pltpu.VMEM((_RPW, 128), jnp.int32),
            pltpu.VMEM((128, _D), jnp.float32),
            pltpu.VMEM((128, _D), jnp.float32),
            pltpu.VMEM((_KC,), jnp.float32),
            pltpu.SemaphoreType.DMA,
            pltpu.SemaphoreType.DMA,
        ],
        compiler_params=pltpu.CompilerParams(needs_layout_passes=False),
    )


def kernel(K, V, cb_k, cb_v, step):
    zk = K.reshape(_TOK, _D)
    zv = V.reshape(_TOK, _D)
    c2_k = jnp.sum(cb_k ** 2, axis=1)[:, None]
    c2_v = jnp.sum(cb_v ** 2, axis=1)[:, None]
    z2_k = jnp.sum(zk ** 2, axis=1).reshape(_NT, 1, _T)
    z2_v = jnp.sum(zv ** 2, axis=1).reshape(_NT, 1, _T)
    idx_k, loss_k = _nearest(zk, cb_k * -2.0, c2_k, z2_k)
    zq_k, cnt_k = _gather_count()(cb_k, idx_k)
    idx_vv, loss_v = _nearest(zv, cb_v * -2.0, c2_v, z2_v)
    zq_v, cnt_v = _gather_count()(cb_v, idx_vv)
    K_mix = zq_k.reshape(K.shape)
    V_mix = zq_v.reshape(V.shape)
    denom = float(_TOK * _D)
    lk = jnp.sum(loss_k) / denom
    lv = jnp.sum(loss_v) / denom
    usage_k = jnp.sum(cnt_k, axis=0) / float(_TOK)
    usage_v = jnp.sum(cnt_v, axis=0) / float(_TOK)
    return (K_mix, V_mix, 0.25 * lk, 0.25 * lv, 0.25 * lk, 0.25 * lv,
            usage_k, usage_v)


# speedup vs baseline: 1.5967x; 1.1965x over previous
import functools

import jax
import jax.numpy as jnp
from jax import lax
from jax.experimental import pallas as pl
from jax.experimental.pallas import tpu as pltpu
from jax.experimental.pallas import tpu_sc as plsc

_KC = 1024
_D = 128
_TOK = 2 * 16 * 2048
_T = 1024
_NT = _TOK // _T
_NW = 32
_TPW = _TOK // _NW
_RPW = _TPW // 128


def _dist_body(z_ref, cb2_ref, c2_ref, z2_ref, idx_ref, loss_ref):
    z = z_ref[...]
    cb2 = cb2_ref[...]
    logits2 = lax.dot_general(cb2, z, (((1,), (1,)), ((), ())),
                              preferred_element_type=jnp.float32)
    z2 = z2_ref[0]
    c2 = c2_ref[...]
    dist = z2 + c2 + logits2
    m = jnp.min(dist, axis=0, keepdims=True)
    iota = lax.broadcasted_iota(jnp.int32, (_KC, _T), 0)
    idx_ref[0] = jnp.min(jnp.where(dist <= m, iota, _KC),
                         axis=0, keepdims=True)
    loss_ref[0, 0, 0] = jnp.sum(m)


def _nearest(z_flat, cb2, c2, z2):
    idx, loss = pl.pallas_call(
        _dist_body,
        grid=(_NT,),
        in_specs=[
            pl.BlockSpec((_T, _D), lambda t: (t, 0)),
            pl.BlockSpec((_KC, _D), lambda t: (0, 0)),
            pl.BlockSpec((_KC, 1), lambda t: (0, 0)),
            pl.BlockSpec((1, 1, _T), lambda t: (t, 0, 0)),
        ],
        out_specs=[
            pl.BlockSpec((1, 1, _T), lambda t: (t, 0, 0)),
            pl.BlockSpec((1, 1, 1), lambda t: (t, 0, 0),
                         memory_space=pltpu.SMEM),
        ],
        out_shape=[
            jax.ShapeDtypeStruct((_NT, 1, _T), jnp.int32),
            jax.ShapeDtypeStruct((_NT, 1, 1), jnp.float32),
        ],
    )(z_flat, cb2, c2, z2)
    return idx.reshape(_TOK // 128, 128), loss


def _gather_count_body(cb_ref, idx_ref, zq_ref, cnt_ref,
                       idx_v, rows0_v, rows1_v, cnt_v, sem0, sem1):
    c = lax.axis_index("c")
    s = lax.axis_index("s")
    wid = s * 2 + c
    base = wid * _TPW
    pltpu.sync_copy(idx_ref.at[pl.ds(wid * _RPW, _RPW)], idx_v)

    def _zero(i, carry):
        cnt_v[pl.ds(i * 16, 16)] = jnp.zeros((16,), jnp.float32)
        return carry

    lax.fori_loop(0, _KC // 16, _zero, 0)

    ones = jnp.ones((16,), jnp.float32)

    def _count_row(j):
        def _cnt(k, inner):
            iv = idx_v[j, pl.ds(k * 16, 16)]
            plsc.addupdate_scatter(cnt_v, [iv], ones)
            return inner

        lax.fori_loop(0, 8, _cnt, 0)

    def _row_pair(g, carry):
        j0 = 2 * g
        j1 = 2 * g + 1
        cp0 = pltpu.async_copy(cb_ref.at[idx_v.at[j0]], rows0_v, sem0)
        cp1 = pltpu.async_copy(cb_ref.at[idx_v.at[j1]], rows1_v, sem1)
        cp0.wait()
        pltpu.sync_copy(rows0_v, zq_ref.at[pl.ds(base + j0 * 128, 128)])
        cp1.wait()
        pltpu.sync_copy(rows1_v, zq_ref.at[pl.ds(base + j1 * 128, 128)])
        _count_row(j0)
        _count_row(j1)
        return carry

    lax.fori_loop(0, _RPW // 2, _row_pair, 0)
    pltpu.sync_copy(cnt_v, cnt_ref.at[wid])


@functools.cache
def _gather_count():
    mesh = plsc.VectorSubcoreMesh(core_axis_name="c", subcore_axis_name="s")
    return pl.kernel(
        _gather_count_body,
        mesh=mesh,
        out_type=[
            jax.ShapeDtypeStruct((_TOK, _D), jnp.float32),
            jax.ShapeDtypeStruct((_NW, _KC), jnp.float32),
        ],
        scratch_types=[
            pltpu.VMEM((_RPW, 128), jnp.int32),
            pltpu.VMEM((128, _D), jnp.float32),
            pltpu.VMEM((128, _D), jnp.float32),
            pltpu.VMEM((_KC,), jnp.float32),
            pltpu.SemaphoreType.DMA,
            pltpu.SemaphoreType.DMA,
        ],
        compiler_params=pltpu.CompilerParams(needs_layout_passes=False),
    )


def kernel(K, V, cb_k, cb_v, step):
    zk = K.reshape(_TOK, _D)
    zv = V.reshape(_TOK, _D)
    c2_k = jnp.sum(cb_k ** 2, axis=1)[:, None]
    c2_v = jnp.sum(cb_v ** 2, axis=1)[:, None]
    z2_k = jnp.sum(zk ** 2, axis=1).reshape(_NT, 1, _T)
    z2_v = jnp.sum(zv ** 2, axis=1).reshape(_NT, 1, _T)
    idx_k, loss_k = _nearest(zk, cb_k * -2.0, c2_k, z2_k)
    zq_k, cnt_k = _gather_count()(cb_k, idx_k)
    idx_vv, loss_v = _nearest(zv, cb_v * -2.0, c2_v, z2_v)
    zq_v, cnt_v = _gather_count()(cb_v, idx_vv)
    K_mix = zq_k.reshape(K.shape)
    V_mix = zq_v.reshape(V.shape)
    denom = float(_TOK * _D)
    lk = jnp.sum(loss_k) / denom
    lv = jnp.sum(loss_v) / denom
    usage_k = jnp.sum(cnt_k, axis=0) / float(_TOK)
    usage_v = jnp.sum(cnt_v, axis=0) / float(_TOK)
    return (K_mix, V_mix, 0.25 * lk, 0.25 * lv, 0.25 * lk, 0.25 * lv,
            usage_k, usage_v)


# speedup vs baseline: 1.7008x; 1.0652x over previous
import functools

import jax
import jax.numpy as jnp
from jax import lax
from jax.experimental import pallas as pl
from jax.experimental.pallas import tpu as pltpu
from jax.experimental.pallas import tpu_sc as plsc

_KC = 1024
_D = 128
_TOK = 2 * 16 * 2048
_T = 2048
_NT = _TOK // _T
_NW = 32
_TPW = _TOK // _NW
_RPW = _TPW // 128


def _dist_body(z_ref, cb2_ref, c2_ref, z2_ref, idx_ref, loss_ref):
    z = z_ref[...]
    cb2 = cb2_ref[...]
    logits2 = lax.dot_general(cb2, z, (((1,), (1,)), ((), ())),
                              preferred_element_type=jnp.float32)
    z2 = z2_ref[0]
    c2 = c2_ref[...]
    dist = z2 + c2 + logits2
    m = jnp.min(dist, axis=0, keepdims=True)
    iota = lax.broadcasted_iota(jnp.int32, (_KC, _T), 0)
    idx_ref[0] = jnp.min(jnp.where(dist <= m, iota, _KC),
                         axis=0, keepdims=True)
    loss_ref[0, 0, 0] = jnp.sum(m)


def _nearest(z_flat, cb2, c2, z2):
    idx, loss = pl.pallas_call(
        _dist_body,
        grid=(_NT,),
        in_specs=[
            pl.BlockSpec((_T, _D), lambda t: (t, 0)),
            pl.BlockSpec((_KC, _D), lambda t: (0, 0)),
            pl.BlockSpec((_KC, 1), lambda t: (0, 0)),
            pl.BlockSpec((1, 1, _T), lambda t: (t, 0, 0)),
        ],
        out_specs=[
            pl.BlockSpec((1, 1, _T), lambda t: (t, 0, 0)),
            pl.BlockSpec((1, 1, 1), lambda t: (t, 0, 0),
                         memory_space=pltpu.SMEM),
        ],
        out_shape=[
            jax.ShapeDtypeStruct((_NT, 1, _T), jnp.int32),
            jax.ShapeDtypeStruct((_NT, 1, 1), jnp.float32),
        ],
    )(z_flat, cb2, c2, z2)
    return idx.reshape(_TOK // 128, 128), loss


def _gather_count_body(cb_ref, idx_ref, zq_ref, cnt_ref,
                       idx_v, rows0_v, rows1_v, cnt_v, sem0, sem1):
    c = lax.axis_index("c")
    s = lax.axis_index("s")
    wid = s * 2 + c
    base = wid * _TPW
    pltpu.sync_copy(idx_ref.at[pl.ds(wid * _RPW, _RPW)], idx_v)

    def _zero(i, carry):
        cnt_v[pl.ds(i * 16, 16)] = jnp.zeros((16,), jnp.float32)
        return carry

    lax.fori_loop(0, _KC // 16, _zero, 0)

    ones = jnp.ones((16,), jnp.float32)

    def _count_row(j):
        def _cnt(k, inner):
            iv = idx_v[j, pl.ds(k * 16, 16)]
            plsc.addupdate_scatter(cnt_v, [iv], ones)
            return inner

        lax.fori_loop(0, 8, _cnt, 0)

    def _row_pair(g, carry):
        j0 = 2 * g
        j1 = 2 * g + 1
        cp0 = pltpu.async_copy(cb_ref.at[idx_v.at[j0]], rows0_v, sem0)
        cp1 = pltpu.async_copy(cb_ref.at[idx_v.at[j1]], rows1_v, sem1)
        cp0.wait()
        pltpu.sync_copy(rows0_v, zq_ref.at[pl.ds(base + j0 * 128, 128)])
        cp1.wait()
        pltpu.sync_copy(rows1_v, zq_ref.at[pl.ds(base + j1 * 128, 128)])
        _count_row(j0)
        _count_row(j1)
        return carry

    lax.fori_loop(0, _RPW // 2, _row_pair, 0)
    pltpu.sync_copy(cnt_v, cnt_ref.at[wid])


@functools.cache
def _gather_count():
    mesh = plsc.VectorSubcoreMesh(core_axis_name="c", subcore_axis_name="s")
    return pl.kernel(
        _gather_count_body,
        mesh=mesh,
        out_type=[
            jax.ShapeDtypeStruct((_TOK, _D), jnp.float32),
            jax.ShapeDtypeStruct((_NW, _KC), jnp.float32),
        ],
        scratch_types=[
            pltpu.VMEM((_RPW, 128), jnp.int32),
            pltpu.VMEM((128, _D), jnp.float32),
            pltpu.VMEM((128, _D), jnp.float32),
            pltpu.VMEM((_KC,), jnp.float32),
            pltpu.SemaphoreType.DMA,
            pltpu.SemaphoreType.DMA,
        ],
        compiler_params=pltpu.CompilerParams(needs_layout_passes=False),
    )


def kernel(K, V, cb_k, cb_v, step):
    zk = K.reshape(_TOK, _D)
    zv = V.reshape(_TOK, _D)
    c2_k = jnp.sum(cb_k ** 2, axis=1)[:, None]
    c2_v = jnp.sum(cb_v ** 2, axis=1)[:, None]
    z2_k = jnp.sum(zk ** 2, axis=1).reshape(_NT, 1, _T)
    z2_v = jnp.sum(zv ** 2, axis=1).reshape(_NT, 1, _T)
    idx_k, loss_k = _nearest(zk, cb_k * -2.0, c2_k, z2_k)
    zq_k, cnt_k = _gather_count()(cb_k, idx_k)
    idx_vv, loss_v = _nearest(zv, cb_v * -2.0, c2_v, z2_v)
    zq_v, cnt_v = _gather_count()(cb_v, idx_vv)
    K_mix = zq_k.reshape(K.shape)
    V_mix = zq_v.reshape(V.shape)
    denom = float(_TOK * _D)
    lk = jnp.sum(loss_k) / denom
    lv = jnp.sum(loss_v) / denom
    usage_k = jnp.sum(cnt_k, axis=0) / float(_TOK)
    usage_v = jnp.sum(cnt_v, axis=0) / float(_TOK)
    return (K_mix, V_mix, 0.25 * lk, 0.25 * lv, 0.25 * lk, 0.25 * lv,
            usage_k, usage_v)


# speedup vs baseline: 1.7409x; 1.0236x over previous
import functools

import jax
import jax.numpy as jnp
from jax import lax
from jax.experimental import pallas as pl
from jax.experimental.pallas import tpu as pltpu
from jax.experimental.pallas import tpu_sc as plsc

_KC = 1024
_D = 128
_TOK = 2 * 16 * 2048
_T = 4096
_NT = _TOK // _T
_NW = 32
_TPW = _TOK // _NW
_RPW = _TPW // 128


def _dist_body(z_ref, cb2_ref, c2_ref, z2_ref, idx_ref, loss_ref):
    z = z_ref[...]
    cb2 = cb2_ref[...]
    logits2 = lax.dot_general(cb2, z, (((1,), (1,)), ((), ())),
                              preferred_element_type=jnp.float32)
    z2 = z2_ref[0]
    c2 = c2_ref[...]
    dist = z2 + c2 + logits2
    m = jnp.min(dist, axis=0, keepdims=True)
    iota = lax.broadcasted_iota(jnp.int32, (_KC, _T), 0)
    idx_ref[0] = jnp.min(jnp.where(dist <= m, iota, _KC),
                         axis=0, keepdims=True)
    loss_ref[0, 0, 0] = jnp.sum(m)


def _nearest(z_flat, cb2, c2, z2):
    idx, loss = pl.pallas_call(
        _dist_body,
        grid=(_NT,),
        in_specs=[
            pl.BlockSpec((_T, _D), lambda t: (t, 0)),
            pl.BlockSpec((_KC, _D), lambda t: (0, 0)),
            pl.BlockSpec((_KC, 1), lambda t: (0, 0)),
            pl.BlockSpec((1, 1, _T), lambda t: (t, 0, 0)),
        ],
        out_specs=[
            pl.BlockSpec((1, 1, _T), lambda t: (t, 0, 0)),
            pl.BlockSpec((1, 1, 1), lambda t: (t, 0, 0),
                         memory_space=pltpu.SMEM),
        ],
        out_shape=[
            jax.ShapeDtypeStruct((_NT, 1, _T), jnp.int32),
            jax.ShapeDtypeStruct((_NT, 1, 1), jnp.float32),
        ],
    )(z_flat, cb2, c2, z2)
    return idx.reshape(_TOK // 128, 128), loss


def _gather_count_body(cb_ref, idx_ref, zq_ref, cnt_ref,
                       idx_v, rows0_v, rows1_v, cnt_v, sem0, sem1):
    c = lax.axis_index("c")
    s = lax.axis_index("s")
    wid = s * 2 + c
    base = wid * _TPW
    pltpu.sync_copy(idx_ref.at[pl.ds(wid * _RPW, _RPW)], idx_v)

    def _zero(i, carry):
        cnt_v[pl.ds(i * 16, 16)] = jnp.zeros((16,), jnp.float32)
        return carry

    lax.fori_loop(0, _KC // 16, _zero, 0)

    ones = jnp.ones((16,), jnp.float32)

    def _count_row(j):
        def _cnt(k, inner):
            iv = idx_v[j, pl.ds(k * 16, 16)]
            plsc.addupdate_scatter(cnt_v, [iv], ones)
            return inner

        lax.fori_loop(0, 8, _cnt, 0)

    def _row_pair(g, carry):
        j0 = 2 * g
        j1 = 2 * g + 1
        cp0 = pltpu.async_copy(cb_ref.at[idx_v.at[j0]], rows0_v, sem0)
        cp1 = pltpu.async_copy(cb_ref.at[idx_v.at[j1]], rows1_v, sem1)
        cp0.wait()
        pltpu.sync_copy(rows0_v, zq_ref.at[pl.ds(base + j0 * 128, 128)])
        cp1.wait()
        pltpu.sync_copy(rows1_v, zq_ref.at[pl.ds(base + j1 * 128, 128)])
        _count_row(j0)
        _count_row(j1)
        return carry

    lax.fori_loop(0, _RPW // 2, _row_pair, 0)
    pltpu.sync_copy(cnt_v, cnt_ref.at[wid])


@functools.cache
def _gather_count():
    mesh = plsc.VectorSubcoreMesh(core_axis_name="c", subcore_axis_name="s")
    return pl.kernel(
        _gather_count_body,
        mesh=mesh,
        out_type=[
            jax.ShapeDtypeStruct((_TOK, _D), jnp.float32),
            jax.ShapeDtypeStruct((_NW, _KC), jnp.float32),
        ],
        scratch_types=[
            pltpu.VMEM((_RPW, 128), jnp.int32),
            pltpu.VMEM((128, _D), jnp.float32),
            pltpu.VMEM((128, _D), jnp.float32),
            pltpu.VMEM((_KC,), jnp.float32),
            pltpu.SemaphoreType.DMA,
            pltpu.SemaphoreType.DMA,
        ],
        compiler_params=pltpu.CompilerParams(needs_layout_passes=False),
    )


def kernel(K, V, cb_k, cb_v, step):
    zk = K.reshape(_TOK, _D)
    zv = V.reshape(_TOK, _D)
    c2_k = jnp.sum(cb_k ** 2, axis=1)[:, None]
    c2_v = jnp.sum(cb_v ** 2, axis=1)[:, None]
    z2_k = jnp.sum(zk ** 2, axis=1).reshape(_NT, 1, _T)
    z2_v = jnp.sum(zv ** 2, axis=1).reshape(_NT, 1, _T)
    idx_k, loss_k = _nearest(zk, cb_k * -2.0, c2_k, z2_k)
    zq_k, cnt_k = _gather_count()(cb_k, idx_k)
    idx_vv, loss_v = _nearest(zv, cb_v * -2.0, c2_v, z2_v)
    zq_v, cnt_v = _gather_count()(cb_v, idx_vv)
    K_mix = zq_k.reshape(K.shape)
    V_mix = zq_v.reshape(V.shape)
    denom = float(_TOK * _D)
    lk = jnp.sum(loss_k) / denom
    lv = jnp.sum(loss_v) / denom
    usage_k = jnp.sum(cnt_k, axis=0) / float(_TOK)
    usage_v = jnp.sum(cnt_v, axis=0) / float(_TOK)
    return (K_mix, V_mix, 0.25 * lk, 0.25 * lv, 0.25 * lk, 0.25 * lv,
            usage_k, usage_v)


# speedup vs baseline: 1.7566x; 1.0090x over previous
import functools

import jax
import jax.numpy as jnp
from jax import lax
from jax.experimental import pallas as pl
from jax.experimental.pallas import tpu as pltpu
from jax.experimental.pallas import tpu_sc as plsc

_KC = 1024
_D = 128
_TOK = 2 * 16 * 2048
_T = 8192
_NT = _TOK // _T
_NW = 32
_TPW = _TOK // _NW
_RPW = _TPW // 128


def _dist_body(z_ref, cb2_ref, c2_ref, z2_ref, idx_ref, loss_ref):
    z = z_ref[...]
    cb2 = cb2_ref[...]
    logits2 = lax.dot_general(cb2, z, (((1,), (1,)), ((), ())),
                              preferred_element_type=jnp.float32)
    z2 = z2_ref[0]
    c2 = c2_ref[...]
    dist = z2 + c2 + logits2
    m = jnp.min(dist, axis=0, keepdims=True)
    iota = lax.broadcasted_iota(jnp.int32, (_KC, _T), 0)
    idx_ref[0] = jnp.min(jnp.where(dist <= m, iota, _KC),
                         axis=0, keepdims=True)
    loss_ref[0, 0, 0] = jnp.sum(m)


def _nearest(z_flat, cb2, c2, z2):
    idx, loss = pl.pallas_call(
        _dist_body,
        grid=(_NT,),
        in_specs=[
            pl.BlockSpec((_T, _D), lambda t: (t, 0)),
            pl.BlockSpec((_KC, _D), lambda t: (0, 0)),
            pl.BlockSpec((_KC, 1), lambda t: (0, 0)),
            pl.BlockSpec((1, 1, _T), lambda t: (t, 0, 0)),
        ],
        out_specs=[
            pl.BlockSpec((1, 1, _T), lambda t: (t, 0, 0)),
            pl.BlockSpec((1, 1, 1), lambda t: (t, 0, 0),
                         memory_space=pltpu.SMEM),
        ],
        out_shape=[
            jax.ShapeDtypeStruct((_NT, 1, _T), jnp.int32),
            jax.ShapeDtypeStruct((_NT, 1, 1), jnp.float32),
        ],
    )(z_flat, cb2, c2, z2)
    return idx.reshape(_TOK // 128, 128), loss


def _gather_count_body(cb_ref, idx_ref, zq_ref, cnt_ref,
                       idx_v, rows0_v, rows1_v, cnt_v, sem0, sem1):
    c = lax.axis_index("c")
    s = lax.axis_index("s")
    wid = s * 2 + c
    base = wid * _TPW
    pltpu.sync_copy(idx_ref.at[pl.ds(wid * _RPW, _RPW)], idx_v)

    def _zero(i, carry):
        cnt_v[pl.ds(i * 16, 16)] = jnp.zeros((16,), jnp.float32)
        return carry

    lax.fori_loop(0, _KC // 16, _zero, 0)

    ones = jnp.ones((16,), jnp.float32)

    def _count_row(j):
        def _cnt(k, inner):
            iv = idx_v[j, pl.ds(k * 16, 16)]
            plsc.addupdate_scatter(cnt_v, [iv], ones)
            return inner

        lax.fori_loop(0, 8, _cnt, 0)

    def _row_pair(g, carry):
        j0 = 2 * g
        j1 = 2 * g + 1
        cp0 = pltpu.async_copy(cb_ref.at[idx_v.at[j0]], rows0_v, sem0)
        cp1 = pltpu.async_copy(cb_ref.at[idx_v.at[j1]], rows1_v, sem1)
        cp0.wait()
        pltpu.sync_copy(rows0_v, zq_ref.at[pl.ds(base + j0 * 128, 128)])
        cp1.wait()
        pltpu.sync_copy(rows1_v, zq_ref.at[pl.ds(base + j1 * 128, 128)])
        _count_row(j0)
        _count_row(j1)
        return carry

    lax.fori_loop(0, _RPW // 2, _row_pair, 0)
    pltpu.sync_copy(cnt_v, cnt_ref.at[wid])


@functools.cache
def _gather_count():
    mesh = plsc.VectorSubcoreMesh(core_axis_name="c", subcore_axis_name="s")
    return pl.kernel(
        _gather_count_body,
        mesh=mesh,
        out_type=[
            jax.ShapeDtypeStruct((_TOK, _D), jnp.float32),
            jax.ShapeDtypeStruct((_NW, _KC), jnp.float32),
        ],
        scratch_types=[
            pltpu.VMEM((_RPW, 128), jnp.int32),
            pltpu.VMEM((128, _D), jnp.float32),
            pltpu.VMEM((128, _D), jnp.float32),
            pltpu.VMEM((_KC,), jnp.float32),
            pltpu.SemaphoreType.DMA,
            pltpu.SemaphoreType.DMA,
        ],
        compiler_params=pltpu.CompilerParams(needs_layout_passes=False),
    )


def kernel(K, V, cb_k, cb_v, step):
    zk = K.reshape(_TOK, _D)
    zv = V.reshape(_TOK, _D)
    c2_k = jnp.sum(cb_k ** 2, axis=1)[:, None]
    c2_v = jnp.sum(cb_v ** 2, axis=1)[:, None]
    z2_k = jnp.sum(zk ** 2, axis=1).reshape(_NT, 1, _T)
    z2_v = jnp.sum(zv ** 2, axis=1).reshape(_NT, 1, _T)
    idx_k, loss_k = _nearest(zk, cb_k * -2.0, c2_k, z2_k)
    zq_k, cnt_k = _gather_count()(cb_k, idx_k)
    idx_vv, loss_v = _nearest(zv, cb_v * -2.0, c2_v, z2_v)
    zq_v, cnt_v = _gather_count()(cb_v, idx_vv)
    K_mix = zq_k.reshape(K.shape)
    V_mix = zq_v.reshape(V.shape)
    denom = float(_TOK * _D)
    lk = jnp.sum(loss_k) / denom
    lv = jnp.sum(loss_v) / denom
    usage_k = jnp.sum(cnt_k, axis=0) / float(_TOK)
    usage_v = jnp.sum(cnt_v, axis=0) / float(_TOK)
    return (K_mix, V_mix, 0.25 * lk, 0.25 * lv, 0.25 * lk, 0.25 * lv,
            usage_k, usage_v)
